# baseline (speedup 1.0000x reference)
import functools

import jax
import jax.numpy as jnp
from jax import lax
from jax.experimental import pallas as pl
from jax.experimental.pallas import tpu as pltpu
from jax.experimental.pallas import tpu_sc as plsc

RC = 5.0
_BLK = 1280


def _silu(h):
    return h / (1.0 + jnp.exp(-h))


def _edge_stage_body(x_ref, xvt_ref, w1a_ref, b1a_ref, w2a_ref,
                     w1b_ref, b1b_ref, w2b_ref, y_ref, r1_ref, r2_ref):
    r = x_ref[...]
    B = r.shape[0]
    rs = jnp.maximum(r, 1e-2)
    pref = jnp.sqrt(2.0 / RC) / rs
    theta = rs * (jnp.pi / RC)
    s1 = jnp.sin(theta)
    c2 = 2.0 * jnp.cos(theta)
    sines = [s1, c2 * s1 - 0.0]
    sines[1] = c2 * s1
    for _ in range(6):
        sines.append(c2 * sines[-1] - sines[-2])
    rb = jnp.stack(sines, axis=-1) * pref[..., None]

    vx = xvt_ref[:, 0, :]
    vy = xvt_ref[:, 1, :]
    vz = xvt_ref[:, 2, :]
    inv = 1.0 / (jnp.sqrt(vx * vx + vy * vy + vz * vz) + 1e-9)
    xh = vx * inv
    yh = vy * inv
    zh = vz * inv
    parts = [jnp.ones_like(xh), xh, yh, zh,
             xh * yh, yh * zh, 3.0 * zh * zh - 1.0, xh * zh, xh * xh - yh * yh]
    y_ref[...] = jnp.stack(parts + [jnp.zeros_like(xh)] * 7, axis=-1)

    w1a = w1a_ref[...]
    w2a = w2a_ref[...]
    w1b = w1b_ref[...]
    w2b = w2b_ref[...]
    b1a = b1a_ref[...]
    b1b = b1b_ref[...]
    for b in range(B):
        rb_b = rb[b]
        ha = _silu(jnp.dot(rb_b, w1a, preferred_element_type=jnp.float32) + b1a)
        r1_ref[b] = jnp.dot(ha, w2a, preferred_element_type=jnp.float32)
        hb = _silu(jnp.dot(rb_b, w1b, preferred_element_type=jnp.float32) + b1b)
        r2_ref[b] = jnp.dot(hb, w2b, preferred_element_type=jnp.float32)


def _edge_stage(x, x_v, R1_W1, R1_b1, R1_W2, R2_W1, R2_b1, R2_W2):
    B, E = x.shape
    grid = (E // _BLK,)
    full = lambda shape: pl.BlockSpec(shape, lambda i: tuple(0 for _ in shape))
    return pl.pallas_call(
        _edge_stage_body,
        grid=grid,
        in_specs=[
            pl.BlockSpec((B, _BLK), lambda i: (0, i)),
            pl.BlockSpec((B, 3, _BLK), lambda i: (0, 0, i)),
            full((8, 64)), full((1, 64)), full((64, 64)),
            full((8, 64)), full((1, 64)), full((64, 64)),
        ],
        out_specs=[
            pl.BlockSpec((B, _BLK, 16), lambda i: (0, i, 0)),
            pl.BlockSpec((B, _BLK, 64), lambda i: (0, i, 0)),
            pl.BlockSpec((B, _BLK, 64), lambda i: (0, i, 0)),
        ],
        out_shape=[
            jax.ShapeDtypeStruct((B, E, 16), jnp.float32),
            jax.ShapeDtypeStruct((B, E, 64), jnp.float32),
            jax.ShapeDtypeStruct((B, E, 64), jnp.float32),
        ],
    )(x, jnp.swapaxes(x_v, 1, 2), R1_W1, R1_b1.reshape(1, 64), R1_W2, R2_W1, R2_b1.reshape(1, 64), R2_W2)


def _node_stage_body(na_ref, w_ref, h0_ref):
    na = na_ref[...]
    w = w_ref[...]
    for b in range(na.shape[0]):
        h0_ref[b] = jnp.dot(na[b], w, preferred_element_type=jnp.float32)


def _node_stage(node_attr, W_node):
    B, N, Z = node_attr.shape
    NBLK = 2000
    return pl.pallas_call(
        _node_stage_body,
        grid=(N // NBLK,),
        in_specs=[
            pl.BlockSpec((B, NBLK, Z), lambda i: (0, i, 0)),
            pl.BlockSpec((Z, 64), lambda i: (0, 0)),
        ],
        out_specs=pl.BlockSpec((B, NBLK, 64), lambda i: (0, i, 0)),
        out_shape=jax.ShapeDtypeStruct((B, N, 64), jnp.float32),
    )(node_attr, W_node)


def _sc_gather(table, idx, CH):
    ROWS = idx.shape[0]
    D = table.shape[1]
    info = plsc.get_sparse_core_info()
    NC, NS = info.num_cores, info.num_subcores
    NW = NC * NS
    per_w = ROWS // NW
    n_ch = per_w // CH
    mesh = plsc.VectorSubcoreMesh(core_axis_name="c", subcore_axis_name="s")

    @functools.partial(
        pl.kernel, mesh=mesh,
        compiler_params=pltpu.CompilerParams(use_tc_tiling_on_sc=False),
        out_type=jax.ShapeDtypeStruct((ROWS, D), table.dtype),
        scratch_types=[
            pltpu.VMEM((per_w,), jnp.int32),
            pltpu.VMEM((CH, D), table.dtype),
            pltpu.VMEM((CH, D), table.dtype),
            pltpu.SemaphoreType.DMA,
            pltpu.SemaphoreType.DMA,
        ],
    )
    def k(table_hbm, idx_hbm, out_hbm, idx_v, rows0, rows1, sem0, sem1):
        wid = lax.axis_index("s") * NC + lax.axis_index("c")
        base = wid * per_w
        pltpu.sync_copy(idx_hbm.at[pl.ds(base, per_w)], idx_v)
        bufs = (rows0, rows1)
        sems = (sem0, sem1)
        cp0 = pltpu.async_copy(table_hbm.at[idx_v.at[pl.ds(0, CH)]], rows0, sem0)
        del cp0

        def body(j, carry):
            @pl.when(j + 1 < n_ch)
            def _():
                for par in range(2):
                    @pl.when((j + 1) % 2 == par)
                    def _():
                        pltpu.async_copy(
                            table_hbm.at[idx_v.at[pl.ds((j + 1) * CH, CH)]],
                            bufs[par], sems[par])
            for par in range(2):
                @pl.when(j % 2 == par)
                def _():
                    pltpu.make_async_copy(
                        table_hbm.at[idx_v.at[pl.ds(j * CH, CH)]],
                        bufs[par], sems[par]).wait()
                    pltpu.sync_copy(bufs[par], out_hbm.at[pl.ds(base + j * CH, CH)])
            return carry

        lax.fori_loop(0, n_ch, body, 0)

    return k(table, idx)

def _msg_stage_body(g_ref, r1_ref, y_ref, rr_ref, p_ref, q_ref, eph_ref, out1_ref):
    i = pl.program_id(0)
    B = g_ref.shape[0]
    P = p_ref[...]
    Q = q_ref[...]
    rr = rr_ref[...]
    parts = []
    for b in range(B):
        a = g_ref[b] * r1_ref[b]
        y = y_ref[b]
        ybc = jnp.dot(y, P, preferred_element_type=jnp.float32)
        abc = jnp.dot(a, Q, preferred_element_type=jnp.float32)
        eph_ref[b] = (ybc * abc).astype(jnp.bfloat16)
        dot1 = jnp.sum(a * rr, axis=-1)
        parts.append(jnp.dot(dot1[None, :], y, preferred_element_type=jnp.float32))
    part = jnp.concatenate(parts, axis=0)
    @pl.when(i == 0)
    def _():
        out1_ref[...] = jnp.zeros_like(out1_ref)
    out1_ref[...] += part


def _msg_stage(g, R1, Y16, r1_read, P, Q):
    B, E, C = g.shape
    grid = (E // _BLK,)
    return pl.pallas_call(
        _msg_stage_body,
        grid=grid,
        in_specs=[
            pl.BlockSpec((B, _BLK, 64), lambda i: (0, i, 0)),
            pl.BlockSpec((B, _BLK, 64), lambda i: (0, i, 0)),
            pl.BlockSpec((B, _BLK, 16), lambda i: (0, i, 0)),
            pl.BlockSpec((1, 64), lambda i: (0, 0)),
            pl.BlockSpec((16, 576), lambda i: (0, 0)),
            pl.BlockSpec((64, 576), lambda i: (0, 0)),
        ],
        out_specs=[
            pl.BlockSpec((B, _BLK, 576), lambda i: (0, i, 0)),
            pl.BlockSpec((B, 16), lambda i: (0, 0)),
        ],
        out_shape=[
            jax.ShapeDtypeStruct((B, E, 576), jnp.bfloat16),
            jax.ShapeDtypeStruct((B, 16), jnp.float32),
        ],
    )(g, R1, Y16, r1_read.reshape(1, 64), P, Q)


def _pass2_stage_body(G_ref, y_ref, r2_ref, p_ref, w_ref, rr_ref, out2_ref):
    i = pl.program_id(0)
    B = y_ref.shape[0]
    P = p_ref[...]
    W = w_ref[...]
    rr = rr_ref[...]
    parts = []
    for b in range(B):
        y = y_ref[b]
        ybc = jnp.dot(y, P, preferred_element_type=jnp.float32)
        Gb = G_ref[b].astype(jnp.float32)
        scal = jnp.dot(Gb * ybc, W, preferred_element_type=jnp.float32)
        b2 = scal * r2_ref[b]
        dot2 = jnp.sum(b2 * rr, axis=-1)
        parts.append(jnp.dot(dot2[None, :], y, preferred_element_type=jnp.float32))
    part = jnp.concatenate(parts, axis=0)
    @pl.when(i == 0)
    def _():
        out2_ref[...] = jnp.zeros_like(out2_ref)
    out2_ref[...] += part


def _pass2_stage(G, Y16, R2, P, W, r2_read):
    B, E, _ = G.shape
    grid = (E // _BLK,)
    return pl.pallas_call(
        _pass2_stage_body,
        grid=grid,
        in_specs=[
            pl.BlockSpec((B, _BLK, 576), lambda i: (0, i, 0)),
            pl.BlockSpec((B, _BLK, 16), lambda i: (0, i, 0)),
            pl.BlockSpec((B, _BLK, 64), lambda i: (0, i, 0)),
            pl.BlockSpec((16, 576), lambda i: (0, 0)),
            pl.BlockSpec((576, 64), lambda i: (0, 0)),
            pl.BlockSpec((1, 64), lambda i: (0, 0)),
        ],
        out_specs=pl.BlockSpec((B, 16), lambda i: (0, 0)),
        out_shape=jax.ShapeDtypeStruct((B, 16), jnp.float32),
    )(G, Y16, R2, P, W, r2_read.reshape(1, 64))


def kernel(x, x_v, node_attr, edge_index, W_node, R1_W1, R1_b1, R1_W2,
           r1_read, U1, R2_W1, R2_b1, R2_W2, r2_read):
    B, E = x.shape
    N = node_attr.shape[1]

    Y16, R1, R2 = _edge_stage(x, x_v, R1_W1, R1_b1, R1_W2, R2_W1, R2_b1, R2_W2)
    h0 = _node_stage(node_attr, W_node)

    j = jnp.arange(576, dtype=jnp.int32)
    P = (j[None, :] // 64 == jnp.arange(16, dtype=jnp.int32)[:, None]).astype(jnp.float32)
    Q = (j[None, :] % 64 == jnp.arange(64, dtype=jnp.int32)[:, None]).astype(jnp.float32)
    W = jnp.tile(U1, (9, 1))

    src = edge_index[:, 0, :]
    dst = edge_index[:, 1, :]

    offs = (jnp.arange(B, dtype=jnp.int32) * N)[:, None]
    flat_src = (src + offs).reshape(B * E)

    g = _sc_gather(h0.reshape(B * N, 64), flat_src, 80).reshape(B, E, 64)
    eph, out1 = _msg_stage(g, R1, Y16, r1_read, P, Q)
    A = jax.vmap(lambda u, i: jax.ops.segment_sum(u, i, num_segments=N))(eph, dst)
    A_i32 = jax.lax.bitcast_convert_type(
        A.reshape(B * N, 288, 2), jnp.int32)
    G_i32 = _sc_gather(A_i32, flat_src, 80)
    G = jax.lax.bitcast_convert_type(
        G_i32, jnp.bfloat16).reshape(B, E, 576)
    out2 = _pass2_stage(G, Y16, R2, P, W, r2_read)
    return out1[:, :9] + out2[:, :9]
